# baseline (speedup 1.0000x reference)
import jax
import jax.numpy as jnp
from jax import lax
from jax.experimental import pallas as pl
from jax.experimental.pallas import tpu as pltpu
from jax.experimental.pallas import tpu_sc as plsc

_NC = 2
_NS = 16
_NW = _NC * _NS
_L = 16

_D = 64
_CHUNK = 64


def _rsqrt(x):
    i = plsc.bitcast(x, jnp.int32)
    i = jnp.int32(0x5F3759DF) - lax.shift_right_logical(i, 1)
    y = plsc.bitcast(i, jnp.float32)
    for _ in range(3):
        y = y * (1.5 - 0.5 * x * y * y)
    return y


def _body(users, pos_items, neg_items, relations, ent_emb, rel_emb,
          out_pos, out_neg,
          idx_u, idx_p, idx_n, idx_r,
          ru0, rp0, rn0, rr0, ru1, rp1, rn1, rr1,
          outp_v, outn_v, sem0, sem1):
    wid = lax.axis_index("s") * _NC + lax.axis_index("c")
    per_w = out_pos.shape[0] // _NW
    n_chunks = per_w // _CHUNK
    wbase = wid * per_w
    lane_iota = lax.iota(jnp.int32, _L)

    bufs = [(ru0, rp0, rn0, rr0, sem0), (ru1, rp1, rn1, rr1, sem1)]

    def fire(c, bset):
        ru, rp, rn, rr, sem = bset
        cbase = wbase + c * _CHUNK
        pltpu.sync_copy(users.at[pl.ds(cbase, _CHUNK)], idx_u)
        pltpu.sync_copy(pos_items.at[pl.ds(cbase, _CHUNK)], idx_p)
        pltpu.sync_copy(neg_items.at[pl.ds(cbase, _CHUNK)], idx_n)
        pltpu.sync_copy(relations.at[pl.ds(cbase, _CHUNK)], idx_r)

        def fgroup(g, _):
            e0 = g * _L
            gsl = pl.ds(e0, _L)
            vu = idx_u[gsl]
            vp = idx_p[gsl]
            vn = idx_n[gsl]
            vr = idx_r[gsl]
            for j in range(_L):
                e = e0 + j
                pltpu.async_copy(ent_emb.at[pl.ds(vu[j], 1), :],
                                 ru.at[pl.ds(e, 1), :], sem)
                pltpu.async_copy(ent_emb.at[pl.ds(vp[j], 1), :],
                                 rp.at[pl.ds(e, 1), :], sem)
                pltpu.async_copy(ent_emb.at[pl.ds(vn[j], 1), :],
                                 rn.at[pl.ds(e, 1), :], sem)
                pltpu.async_copy(rel_emb.at[pl.ds(vr[j], 1), :],
                                 rr.at[pl.ds(e, 1), :], sem)
            return 0

        lax.fori_loop(0, _CHUNK // _L, fgroup, 0)

    def drain(bset):
        ru, rp, rn, rr, sem = bset
        for dst in (ru, rp, rn, rr):
            pltpu.make_async_copy(ent_emb.at[pl.ds(0, _CHUNK), :], dst,
                                  sem).wait()

    def score(c, bset):
        ru, rp, rn, rr, _ = bset

        def group(g, _):
            resp = jnp.zeros((_L,), jnp.float32)
            resn = jnp.zeros((_L,), jnp.float32)
            for j in range(_L):
                e = g * _L + j
                accp = jnp.zeros((_L,), jnp.float32)
                accn = jnp.zeros((_L,), jnp.float32)
                for k in range(_D // _L):
                    sl = pl.ds(k * _L, _L)
                    u = ru[e, sl]
                    r = rr[e, sl]
                    p = rp[e, sl]
                    n = rn[e, sl]
                    t = u + r
                    dp = t - p
                    dn = t - n
                    accp = accp + dp * dp
                    accn = accn + dn * dn
                lane = lane_iota == j
                resp = jnp.where(lane, jnp.sum(accp), resp)
                resn = jnp.where(lane, jnp.sum(accn), resn)
            gsl = pl.ds(g * _L, _L)
            outp_v[gsl] = -(resp * _rsqrt(jnp.maximum(resp, 1e-30)))
            outn_v[gsl] = -(resn * _rsqrt(jnp.maximum(resn, 1e-30)))
            return 0

        lax.fori_loop(0, _CHUNK // _L, group, 0)
        cbase = wbase + c * _CHUNK
        pltpu.sync_copy(outp_v, out_pos.at[pl.ds(cbase, _CHUNK)])
        pltpu.sync_copy(outn_v, out_neg.at[pl.ds(cbase, _CHUNK)])

    fire(0, bufs[0])
    for c in range(n_chunks):
        if c + 1 < n_chunks:
            fire(c + 1, bufs[(c + 1) % 2])
        drain(bufs[c % 2])
        score(c, bufs[c % 2])


def kernel(users, pos_items, neg_items, relations, ent_emb, rel_emb):
    B = users.shape[0]
    users = users.astype(jnp.int32)
    pos_items = pos_items.astype(jnp.int32)
    neg_items = neg_items.astype(jnp.int32)
    relations = relations.astype(jnp.int32)
    per_w = B // _NW

    nudge = 0.0 * jnp.sum(jnp.take(ent_emb, users, axis=0))

    run = pl.kernel(
        _body,
        out_type=(
            jax.ShapeDtypeStruct((B,), jnp.float32),
            jax.ShapeDtypeStruct((B,), jnp.float32),
        ),
        mesh=plsc.VectorSubcoreMesh(
            core_axis_name="c", subcore_axis_name="s",
            num_cores=_NC, num_subcores=_NS,
        ),
        compiler_params=pltpu.CompilerParams(
            needs_layout_passes=False, use_tc_tiling_on_sc=True,
        ),
        scratch_types=[
            pltpu.VMEM((_CHUNK,), jnp.int32),
            pltpu.VMEM((_CHUNK,), jnp.int32),
            pltpu.VMEM((_CHUNK,), jnp.int32),
            pltpu.VMEM((_CHUNK,), jnp.int32),
            pltpu.VMEM((_CHUNK, _D), jnp.float32),
            pltpu.VMEM((_CHUNK, _D), jnp.float32),
            pltpu.VMEM((_CHUNK, _D), jnp.float32),
            pltpu.VMEM((_CHUNK, _D), jnp.float32),
            pltpu.VMEM((_CHUNK, _D), jnp.float32),
            pltpu.VMEM((_CHUNK, _D), jnp.float32),
            pltpu.VMEM((_CHUNK, _D), jnp.float32),
            pltpu.VMEM((_CHUNK, _D), jnp.float32),
            pltpu.VMEM((_CHUNK,), jnp.float32),
            pltpu.VMEM((_CHUNK,), jnp.float32),
            pltpu.SemaphoreType.DMA,
            pltpu.SemaphoreType.DMA,
        ],
    )
    outp, outn = run(users, pos_items, neg_items, relations, ent_emb, rel_emb)
    return (outp + nudge, outn + nudge)
